# baseline (speedup 1.0000x reference)
import functools

import jax
import jax.numpy as jnp
from jax import lax
from jax.experimental import pallas as pl
from jax.experimental.pallas import tpu as pltpu
from jax.experimental.pallas import tpu_sc as plsc

N = 10000
E = 160000
HID = 256
NPAD = 10016
NB = 32
BR = 313
ACC_R = 320
CAP = 161792
BLKB = 1024
GCH = 64
PBLK = 6400
STG = 1088
ROW_BLK = 2504

_mesh = plsc.VectorSubcoreMesh(core_axis_name="c", subcore_axis_name="s")
_params = pltpu.CompilerParams(needs_layout_passes=False)


def _worker_id():
    return lax.axis_index("s") * 2 + lax.axis_index("c")


def _lane_bcast(v, j):
    return lax.gather(
        v, jnp.full((16, 1), j, jnp.int32),
        lax.GatherDimensionNumbers(offset_dims=(), collapsed_slice_dims=(0,),
                                   start_index_map=(0,)),
        (1,), mode=lax.GatherScatterMode.PROMISE_IN_BOUNDS)


def _prep_body(src_h, dst_h, el_src, el_dl, cnt, sbuf0, dbuf0, sbuf1, dbuf1,
               st_src, st_dl, cbuf, sem0, sem1):
    w = _worker_id()
    lo = w * BR
    nblk = E // PBLK
    bufs = ((sbuf0, dbuf0, sem0), (sbuf1, dbuf1, sem1))

    def fire(g, sb, db, sem):
        off = pl.multiple_of(g * PBLK, 8)
        pltpu.async_copy(src_h.at[pl.ds(off, PBLK)], sb, sem)
        pltpu.async_copy(dst_h.at[pl.ds(off, PBLK)], db, sem)

    def make_inner(sb, db):
        def half(off, ptr_v):
            d16 = db[pl.ds(off, 16)]
            s16 = sb[pl.ds(off, 16)]
            msk = (d16 >= lo) & (d16 < lo + BR)
            csum = plsc.cumsum(msk.astype(jnp.int32))
            pos = ptr_v + csum - 1
            plsc.store_scatter(st_src, [pos], s16, mask=msk)
            plsc.store_scatter(st_dl, [pos], d16 - lo, mask=msk)
            return ptr_v + _lane_bcast(csum, 15)

        def inner(c, carry):
            ptr_v, hptr = carry
            off = pl.multiple_of(c * 32, 8)
            ptr_v = half(off, ptr_v)
            ptr_v = half(off + 16, ptr_v)

            def do_flush(args):
                ptr_v, hptr = args
                hoff = pl.multiple_of(w * CAP + hptr, 8)
                pltpu.sync_copy(st_src.at[pl.ds(0, 1024)],
                                el_src.at[pl.ds(hoff, 1024)])
                pltpu.sync_copy(st_dl.at[pl.ds(0, 1024)],
                                el_dl.at[pl.ds(hoff, 1024)])
                st_src[pl.ds(0, 16)] = st_src[pl.ds(1024, 16)]
                st_dl[pl.ds(0, 16)] = st_dl[pl.ds(1024, 16)]
                st_src[pl.ds(16, 16)] = st_src[pl.ds(1040, 16)]
                st_dl[pl.ds(16, 16)] = st_dl[pl.ds(1040, 16)]
                return ptr_v - 1024, hptr + 1024

            return lax.cond(jnp.any(ptr_v >= 1024), do_flush, lambda a: a,
                            (ptr_v, hptr))

        return inner

    fire(0, *bufs[0])
    carry = (jnp.zeros((16,), jnp.int32), jnp.int32(0))
    for g in range(nblk):
        sb, db, sem = bufs[g % 2]
        pltpu.make_async_copy(src_h.at[pl.ds(0, PBLK)], sb, sem).wait()
        pltpu.make_async_copy(dst_h.at[pl.ds(0, PBLK)], db, sem).wait()
        if g + 1 < nblk:
            fire(g + 1, *bufs[(g + 1) % 2])
        carry = pl.loop(0, PBLK // 32, init_carry=carry)(make_inner(sb, db))
    ptr_v, hptr = carry
    ptr = ptr_v[0]

    for t in range(4):
        st_src[pl.ds(ptr + t * 16, 16)] = jnp.zeros((16,), jnp.int32)
        st_dl[pl.ds(ptr + t * 16, 16)] = jnp.full((16,), BR, jnp.int32)
    hoff = pl.multiple_of(w * CAP + hptr, 8)
    pltpu.sync_copy(st_src.at[pl.ds(0, STG)], el_src.at[pl.ds(hoff, STG)])
    pltpu.sync_copy(st_dl.at[pl.ds(0, STG)], el_dl.at[pl.ds(hoff, STG)])
    cbuf[...] = jnp.broadcast_to(hptr + ptr, (16,))
    pltpu.sync_copy(cbuf, cnt.at[pl.ds(pl.multiple_of(w * 16, 8), 16)])


_prep = functools.partial(
    pl.kernel,
    compiler_params=_params,
    out_type=(
        jax.ShapeDtypeStruct((NB * CAP,), jnp.int32),
        jax.ShapeDtypeStruct((NB * CAP,), jnp.int32),
        jax.ShapeDtypeStruct((NB * 16,), jnp.int32),
    ),
    mesh=_mesh,
    scratch_types=[
        pltpu.VMEM((PBLK,), jnp.int32),
        pltpu.VMEM((PBLK,), jnp.int32),
        pltpu.VMEM((PBLK,), jnp.int32),
        pltpu.VMEM((PBLK,), jnp.int32),
        pltpu.VMEM((STG,), jnp.int32),
        pltpu.VMEM((STG,), jnp.int32),
        pltpu.VMEM((16,), jnp.int32),
        pltpu.SemaphoreType.DMA,
        pltpu.SemaphoreType.DMA,
    ],
)(_prep_body)


def _agg_body(d, h, el_src, el_dl, cnt, zeros, m, src_blk, dl_blk, rows0,
              rows1, cbuf, acc, sem0, sem1):
    w = _worker_id()
    iota = lax.iota(jnp.int32, 16)
    pltpu.sync_copy(zeros, acc)
    pltpu.sync_copy(cnt.at[pl.ds(pl.multiple_of(w * 16, 8), 16)], cbuf)
    n = jnp.max(cbuf[...])
    nch = (n + (GCH - 1)) // GCH
    cpb = BLKB // GCH

    def fire(c, buf, sem):
        idx = src_blk.at[pl.ds(pl.multiple_of(c * GCH, 8), GCH)]
        pltpu.async_copy(h.at[idx], buf, sem)

    def wait(buf, sem):
        pltpu.make_async_copy(h.at[src_blk.at[pl.ds(0, GCH)]], buf, sem).wait()

    def process(buf, c):
        def edge(e):
            lane = e % 16
            voff = pl.multiple_of(c * GCH + (e // 16) * 16, 8)
            vec = dl_blk[pl.ds(voff, 16)]
            bc = lax.gather(
                vec, jnp.broadcast_to(lane, (16, 1)),
                lax.GatherDimensionNumbers(offset_dims=(),
                                           collapsed_slice_dims=(0,),
                                           start_index_map=(0,)),
                (1,), mode=lax.GatherScatterMode.PROMISE_IN_BOUNDS)
            rowbase2 = bc * d + iota * 2
            for k in range(d // 32):
                v16 = buf[e, pl.ds(k * 16, 16)]
                va = plsc.bitcast(v16 << 16, jnp.float32)
                vb = plsc.bitcast(v16 & jnp.int32(-65536), jnp.float32)
                plsc.addupdate_scatter(acc, [rowbase2 + (k * 32)], va)
                plsc.addupdate_scatter(acc, [rowbase2 + (k * 32 + 1)], vb)

        plsc.parallel_loop(0, GCH, 1, unroll=2)(edge)

    def outer(g):
        boff = pl.multiple_of(w * CAP + g * BLKB, 8)
        pltpu.sync_copy(el_src.at[pl.ds(boff, BLKB)], src_blk)
        pltpu.sync_copy(el_dl.at[pl.ds(boff, BLKB)], dl_blk)
        ch_here = jnp.minimum(cpb, nch - g * cpb)
        fire(0, rows0, sem0)

        def pair(p):
            c0 = p * 2
            c1 = c0 + 1
            wait(rows0, sem0)
            pl.when(c1 < ch_here)(lambda: fire(c1, rows1, sem1))
            process(rows0, c0)

            @pl.when(c1 < ch_here)
            def _():
                wait(rows1, sem1)
                pl.when(c1 + 1 < ch_here)(lambda: fire(c1 + 1, rows0, sem0))
                process(rows1, c1)

        pl.loop(0, (ch_here + 1) // 2)(pair)

    pl.loop(0, (nch + cpb - 1) // cpb)(outer)
    doff = pl.multiple_of(w * BR * d, 8)
    pltpu.sync_copy(acc.at[pl.ds(0, BR * d)], m.at[pl.ds(doff, BR * d)])


def _make_agg(d):
    return functools.partial(
        pl.kernel,
        compiler_params=_params,
        out_type=jax.ShapeDtypeStruct((NPAD * d,), jnp.float32),
        mesh=_mesh,
        scratch_types=[
            pltpu.VMEM((BLKB,), jnp.int32),
            pltpu.VMEM((BLKB,), jnp.int32),
            pltpu.VMEM((GCH, d // 2), jnp.int32),
            pltpu.VMEM((GCH, d // 2), jnp.int32),
            pltpu.VMEM((16,), jnp.int32),
            pltpu.VMEM((ACC_R * d,), jnp.float32),
            pltpu.SemaphoreType.DMA,
            pltpu.SemaphoreType.DMA,
        ],
    )(functools.partial(_agg_body, d))


_agg256 = _make_agg(HID)


def _dense_body(m_ref, h_ref, wr_ref, wt_ref, b_ref, o_ref, ob_ref, *, relu,
                rel_mm):
    if rel_mm:
        acc = jnp.dot(m_ref[...], wr_ref[...], preferred_element_type=jnp.float32)
    else:
        acc = m_ref[...]
    acc = acc + jnp.dot(h_ref[...], wt_ref[...], preferred_element_type=jnp.float32)
    acc = acc + b_ref[...]
    if relu:
        acc = jnp.maximum(acc, 0.0)
    o_ref[...] = acc
    if ob_ref is not None:
        ob_ref[...] = acc.astype(jnp.bfloat16)


def _dense_layer(m, h, W_rel, W_root, b, relu, rel_mm=True, emit_bf16=True):
    n, k = h.shape
    mk = m.shape[1]
    out_shape = [jax.ShapeDtypeStruct((n, HID), jnp.float32)]
    out_specs = [pl.BlockSpec((ROW_BLK, HID), lambda i: (i, 0))]
    body = functools.partial(_dense_body, relu=relu, rel_mm=rel_mm)
    if emit_bf16:
        out_shape.append(jax.ShapeDtypeStruct((n, HID), jnp.bfloat16))
        out_specs.append(pl.BlockSpec((ROW_BLK, HID), lambda i: (i, 0)))
    else:
        body = functools.partial(_dense_body, relu=relu, rel_mm=rel_mm,
                                 ob_ref=None)
    out = pl.pallas_call(
        body,
        grid=(n // ROW_BLK,),
        in_specs=[
            pl.BlockSpec((ROW_BLK, mk), lambda i: (i, 0)),
            pl.BlockSpec((ROW_BLK, k), lambda i: (i, 0)),
            pl.BlockSpec((mk, HID), lambda i: (0, 0)),
            pl.BlockSpec((k, HID), lambda i: (0, 0)),
            pl.BlockSpec((1, HID), lambda i: (0, 0)),
        ],
        out_specs=out_specs,
        out_shape=out_shape,
    )(m, h, W_rel, W_root, b.reshape(1, HID))
    return out if emit_bf16 else out[0]


def _mm_body(h_ref, w_ref, o_ref):
    acc = jnp.dot(h_ref[...], w_ref[...], preferred_element_type=jnp.float32)
    o_ref[...] = acc.astype(jnp.bfloat16)


def _matmul_bf16(h, w):
    n, k = h.shape
    return pl.pallas_call(
        _mm_body,
        grid=(n // ROW_BLK,),
        in_specs=[
            pl.BlockSpec((ROW_BLK, k), lambda i: (i, 0)),
            pl.BlockSpec((k, HID), lambda i: (0, 0)),
        ],
        out_specs=pl.BlockSpec((ROW_BLK, HID), lambda i: (i, 0)),
        out_shape=jax.ShapeDtypeStruct((n, HID), jnp.bfloat16),
    )(h, w)


def _pack32(a):
    n, c = a.shape
    return jax.lax.bitcast_convert_type(a.reshape(n, c // 2, 2), jnp.int32)


def kernel(x, edge_index, W_rel0, b_rel0, W_root0, W_rel1, b_rel1, W_root1,
           W_rel2, b_rel2, W_root2, W_rel3, b_rel3, W_root3):
    h0 = jnp.zeros((NPAD, 8), jnp.float32).at[:N, :6].set(x[:, 4:10])
    wr0 = jnp.zeros((8, HID), jnp.float32).at[:6].set(W_rel0)
    wt0 = jnp.zeros((8, HID), jnp.float32).at[:6].set(W_root0)
    z256 = jnp.zeros((ACC_R * HID,), jnp.float32)

    el_src, el_dl, cnt = _prep(edge_index[0], edge_index[1])

    t0 = _pack32(_matmul_bf16(h0, wr0))
    m = _agg256(t0, el_src, el_dl, cnt, z256).reshape(NPAD, HID)
    h, hb = _dense_layer(m, h0, wr0, wt0, b_rel0, relu=True, rel_mm=False)
    for W_rel, b_rel, W_root, relu, last in (
        (W_rel1, b_rel1, W_root1, True, False),
        (W_rel2, b_rel2, W_root2, True, False),
        (W_rel3, b_rel3, W_root3, False, True),
    ):
        m = _agg256(_pack32(hb), el_src, el_dl, cnt, z256).reshape(NPAD, HID)
        if last:
            h = _dense_layer(m, h, W_rel, W_root, b_rel, relu=False,
                             emit_bf16=False)
        else:
            h, hb = _dense_layer(m, h, W_rel, W_root, b_rel, relu=True)
    return h[:N]


# speedup vs baseline: 1.3520x; 1.3520x over previous
import functools

import jax
import jax.numpy as jnp
from jax import lax
from jax.experimental import pallas as pl
from jax.experimental.pallas import tpu as pltpu
from jax.experimental.pallas import tpu_sc as plsc

N = 10000
E = 160000
HID = 256
NPAD = 10016
NB = 32
BR = 313
ACC_R = 320
CAP = 161792
BLKB = 1024
GCH = 64
PBLK = 6400
STG = 1088
ROW_BLK = 2504

_mesh = plsc.VectorSubcoreMesh(core_axis_name="c", subcore_axis_name="s")
_params = pltpu.CompilerParams(needs_layout_passes=False)


def _worker_id():
    return lax.axis_index("s") * 2 + lax.axis_index("c")


def _lane_bcast(v, j):
    return lax.gather(
        v, jnp.full((16, 1), j, jnp.int32),
        lax.GatherDimensionNumbers(offset_dims=(), collapsed_slice_dims=(0,),
                                   start_index_map=(0,)),
        (1,), mode=lax.GatherScatterMode.PROMISE_IN_BOUNDS)


def _prep_body(src_h, dst_h, el_src, el_dl, cnt, sbuf0, dbuf0, sbuf1, dbuf1,
               st_src, st_dl, cbuf, sem0, sem1):
    w = _worker_id()
    lo = w * BR
    nblk = E // PBLK
    bufs = ((sbuf0, dbuf0, sem0), (sbuf1, dbuf1, sem1))

    def fire(g, sb, db, sem):
        off = pl.multiple_of(g * PBLK, 8)
        pltpu.async_copy(src_h.at[pl.ds(off, PBLK)], sb, sem)
        pltpu.async_copy(dst_h.at[pl.ds(off, PBLK)], db, sem)

    def make_inner(sb, db):
        def half(off, ptr_v):
            d16 = db[pl.ds(off, 16)]
            s16 = sb[pl.ds(off, 16)]
            msk = (d16 >= lo) & (d16 < lo + BR)
            csum = plsc.cumsum(msk.astype(jnp.int32))
            pos = ptr_v + csum - 1
            plsc.store_scatter(st_src, [pos], s16, mask=msk)
            plsc.store_scatter(st_dl, [pos], d16 - lo, mask=msk)
            return ptr_v + _lane_bcast(csum, 15)

        def inner(c, carry):
            ptr_v, hptr = carry
            off = pl.multiple_of(c * 32, 8)
            ptr_v = half(off, ptr_v)
            ptr_v = half(off + 16, ptr_v)

            def do_flush(args):
                ptr_v, hptr = args
                hoff = pl.multiple_of(w * CAP + hptr, 8)
                pltpu.sync_copy(st_src.at[pl.ds(0, 1024)],
                                el_src.at[pl.ds(hoff, 1024)])
                pltpu.sync_copy(st_dl.at[pl.ds(0, 1024)],
                                el_dl.at[pl.ds(hoff, 1024)])
                st_src[pl.ds(0, 16)] = st_src[pl.ds(1024, 16)]
                st_dl[pl.ds(0, 16)] = st_dl[pl.ds(1024, 16)]
                st_src[pl.ds(16, 16)] = st_src[pl.ds(1040, 16)]
                st_dl[pl.ds(16, 16)] = st_dl[pl.ds(1040, 16)]
                return ptr_v - 1024, hptr + 1024

            return lax.cond(jnp.any(ptr_v >= 1024), do_flush, lambda a: a,
                            (ptr_v, hptr))

        return inner

    fire(0, *bufs[0])
    carry = (jnp.zeros((16,), jnp.int32), jnp.int32(0))
    for g in range(nblk):
        sb, db, sem = bufs[g % 2]
        pltpu.make_async_copy(src_h.at[pl.ds(0, PBLK)], sb, sem).wait()
        pltpu.make_async_copy(dst_h.at[pl.ds(0, PBLK)], db, sem).wait()
        if g + 1 < nblk:
            fire(g + 1, *bufs[(g + 1) % 2])
        carry = pl.loop(0, PBLK // 32, init_carry=carry)(make_inner(sb, db))
    ptr_v, hptr = carry
    ptr = ptr_v[0]

    for t in range(4):
        st_src[pl.ds(ptr + t * 16, 16)] = jnp.zeros((16,), jnp.int32)
        st_dl[pl.ds(ptr + t * 16, 16)] = jnp.full((16,), BR, jnp.int32)
    hoff = pl.multiple_of(w * CAP + hptr, 8)
    pltpu.sync_copy(st_src.at[pl.ds(0, STG)], el_src.at[pl.ds(hoff, STG)])
    pltpu.sync_copy(st_dl.at[pl.ds(0, STG)], el_dl.at[pl.ds(hoff, STG)])
    cbuf[...] = jnp.broadcast_to(hptr + ptr, (16,))
    pltpu.sync_copy(cbuf, cnt.at[pl.ds(pl.multiple_of(w * 16, 8), 16)])


_prep = functools.partial(
    pl.kernel,
    compiler_params=_params,
    out_type=(
        jax.ShapeDtypeStruct((NB * CAP,), jnp.int32),
        jax.ShapeDtypeStruct((NB * CAP,), jnp.int32),
        jax.ShapeDtypeStruct((NB * 16,), jnp.int32),
    ),
    mesh=_mesh,
    scratch_types=[
        pltpu.VMEM((PBLK,), jnp.int32),
        pltpu.VMEM((PBLK,), jnp.int32),
        pltpu.VMEM((PBLK,), jnp.int32),
        pltpu.VMEM((PBLK,), jnp.int32),
        pltpu.VMEM((STG,), jnp.int32),
        pltpu.VMEM((STG,), jnp.int32),
        pltpu.VMEM((16,), jnp.int32),
        pltpu.SemaphoreType.DMA,
        pltpu.SemaphoreType.DMA,
    ],
)(_prep_body)


def _agg_body(d, h, el_src, el_dl, cnt, zeros, m, src_blk, dl_blk, rows0,
              rows1, cbuf, acc, sem0, sem1):
    w = _worker_id()
    iota = lax.iota(jnp.int32, 16)
    pltpu.sync_copy(zeros, acc)
    pltpu.sync_copy(cnt.at[pl.ds(pl.multiple_of(w * 16, 8), 16)], cbuf)
    n = jnp.max(cbuf[...])
    nch = (n + (GCH - 1)) // GCH
    cpb = BLKB // GCH

    def fire(c, buf, sem):
        idx = src_blk.at[pl.ds(pl.multiple_of(c * GCH, 8), GCH)]
        pltpu.async_copy(h.at[idx], buf, sem)

    def wait(buf, sem):
        pltpu.make_async_copy(h.at[src_blk.at[pl.ds(0, GCH)]], buf, sem).wait()

    def process(buf, c):
        def edge(e):
            lane = e % 16
            voff = pl.multiple_of(c * GCH + (e // 16) * 16, 8)
            vec = dl_blk[pl.ds(voff, 16)]
            bc = lax.gather(
                vec, jnp.broadcast_to(lane, (16, 1)),
                lax.GatherDimensionNumbers(offset_dims=(),
                                           collapsed_slice_dims=(0,),
                                           start_index_map=(0,)),
                (1,), mode=lax.GatherScatterMode.PROMISE_IN_BOUNDS)
            rowbase = bc * d + iota
            for k in range(d // 32):
                v16 = buf[e, pl.ds(k * 16, 16)]
                va = plsc.bitcast(v16 << 16, jnp.float32)
                vb = plsc.bitcast(v16 & jnp.int32(-65536), jnp.float32)
                plsc.addupdate_scatter(acc, [rowbase + (k * 16)], va)
                plsc.addupdate_scatter(acc, [rowbase + (128 + k * 16)], vb)

        plsc.parallel_loop(0, GCH, 1, unroll=4)(edge)

    def outer(g):
        boff = pl.multiple_of(w * CAP + g * BLKB, 8)
        pltpu.sync_copy(el_src.at[pl.ds(boff, BLKB)], src_blk)
        pltpu.sync_copy(el_dl.at[pl.ds(boff, BLKB)], dl_blk)
        ch_here = jnp.minimum(cpb, nch - g * cpb)
        fire(0, rows0, sem0)

        def pair(p):
            c0 = p * 2
            c1 = c0 + 1
            wait(rows0, sem0)
            pl.when(c1 < ch_here)(lambda: fire(c1, rows1, sem1))
            process(rows0, c0)

            @pl.when(c1 < ch_here)
            def _():
                wait(rows1, sem1)
                pl.when(c1 + 1 < ch_here)(lambda: fire(c1 + 1, rows0, sem0))
                process(rows1, c1)

        pl.loop(0, (ch_here + 1) // 2)(pair)

    pl.loop(0, (nch + cpb - 1) // cpb)(outer)
    doff = pl.multiple_of(w * BR * d, 8)
    pltpu.sync_copy(acc.at[pl.ds(0, BR * d)], m.at[pl.ds(doff, BR * d)])


def _make_agg(d):
    return functools.partial(
        pl.kernel,
        compiler_params=_params,
        out_type=jax.ShapeDtypeStruct((NPAD * d,), jnp.float32),
        mesh=_mesh,
        scratch_types=[
            pltpu.VMEM((BLKB,), jnp.int32),
            pltpu.VMEM((BLKB,), jnp.int32),
            pltpu.VMEM((GCH, d // 2), jnp.int32),
            pltpu.VMEM((GCH, d // 2), jnp.int32),
            pltpu.VMEM((16,), jnp.int32),
            pltpu.VMEM((ACC_R * d,), jnp.float32),
            pltpu.SemaphoreType.DMA,
            pltpu.SemaphoreType.DMA,
        ],
    )(functools.partial(_agg_body, d))


_agg256 = _make_agg(HID)


def _pack_words(acc):
    u = jax.lax.bitcast_convert_type(acc, jnp.int32)
    lsb = jax.lax.shift_right_logical(u, 16) & 1
    rb = jax.lax.shift_right_logical(u + 0x7FFF + lsb, 16)
    return rb[:, :128] | (rb[:, 128:] << 16)


def _dense_body(m_ref, h_ref, wr_ref, wt_ref, b_ref, o_ref, ob_ref, *, relu,
                rel_mm):
    if rel_mm:
        acc = jnp.dot(m_ref[...], wr_ref[...], preferred_element_type=jnp.float32)
    else:
        acc = m_ref[...]
    acc = acc + jnp.dot(h_ref[...], wt_ref[...], preferred_element_type=jnp.float32)
    acc = acc + b_ref[...]
    if relu:
        acc = jnp.maximum(acc, 0.0)
    o_ref[...] = acc
    if ob_ref is not None:
        ob_ref[...] = _pack_words(acc)


def _dense_layer(m, h, W_rel, W_root, b, relu, rel_mm=True, emit_bf16=True):
    n, k = h.shape
    mk = m.shape[1]
    out_shape = [jax.ShapeDtypeStruct((n, HID), jnp.float32)]
    out_specs = [pl.BlockSpec((ROW_BLK, HID), lambda i: (i, 0))]
    body = functools.partial(_dense_body, relu=relu, rel_mm=rel_mm)
    if emit_bf16:
        out_shape.append(jax.ShapeDtypeStruct((n, HID // 2), jnp.int32))
        out_specs.append(pl.BlockSpec((ROW_BLK, HID // 2), lambda i: (i, 0)))
    else:
        body = functools.partial(_dense_body, relu=relu, rel_mm=rel_mm,
                                 ob_ref=None)
    out = pl.pallas_call(
        body,
        grid=(n // ROW_BLK,),
        in_specs=[
            pl.BlockSpec((ROW_BLK, mk), lambda i: (i, 0)),
            pl.BlockSpec((ROW_BLK, k), lambda i: (i, 0)),
            pl.BlockSpec((mk, HID), lambda i: (0, 0)),
            pl.BlockSpec((k, HID), lambda i: (0, 0)),
            pl.BlockSpec((1, HID), lambda i: (0, 0)),
        ],
        out_specs=out_specs,
        out_shape=out_shape,
    )(m, h, W_rel, W_root, b.reshape(1, HID))
    return out if emit_bf16 else out[0]


def _mm_body(h_ref, w_ref, o_ref):
    acc = jnp.dot(h_ref[...], w_ref[...], preferred_element_type=jnp.float32)
    o_ref[...] = _pack_words(acc)


def _matmul_bf16(h, w):
    n, k = h.shape
    return pl.pallas_call(
        _mm_body,
        grid=(n // ROW_BLK,),
        in_specs=[
            pl.BlockSpec((ROW_BLK, k), lambda i: (i, 0)),
            pl.BlockSpec((k, HID), lambda i: (0, 0)),
        ],
        out_specs=pl.BlockSpec((ROW_BLK, HID // 2), lambda i: (i, 0)),
        out_shape=jax.ShapeDtypeStruct((n, HID // 2), jnp.int32),
    )(h, w)


def kernel(x, edge_index, W_rel0, b_rel0, W_root0, W_rel1, b_rel1, W_root1,
           W_rel2, b_rel2, W_root2, W_rel3, b_rel3, W_root3):
    h0 = jnp.zeros((NPAD, 8), jnp.float32).at[:N, :6].set(x[:, 4:10])
    wr0 = jnp.zeros((8, HID), jnp.float32).at[:6].set(W_rel0)
    wt0 = jnp.zeros((8, HID), jnp.float32).at[:6].set(W_root0)
    z256 = jnp.zeros((ACC_R * HID,), jnp.float32)

    el_src, el_dl, cnt = _prep(edge_index[0], edge_index[1])

    t0 = _matmul_bf16(h0, wr0)
    m = _agg256(t0, el_src, el_dl, cnt, z256).reshape(NPAD, HID)
    h, hb = _dense_layer(m, h0, wr0, wt0, b_rel0, relu=True, rel_mm=False)
    for W_rel, b_rel, W_root, relu, last in (
        (W_rel1, b_rel1, W_root1, True, False),
        (W_rel2, b_rel2, W_root2, True, False),
        (W_rel3, b_rel3, W_root3, False, True),
    ):
        m = _agg256(hb, el_src, el_dl, cnt, z256).reshape(NPAD, HID)
        if last:
            h = _dense_layer(m, h, W_rel, W_root, b_rel, relu=False,
                             emit_bf16=False)
        else:
            h, hb = _dense_layer(m, h, W_rel, W_root, b_rel, relu=True)
    return h[:N]


# speedup vs baseline: 1.5309x; 1.1324x over previous
import functools

import jax
import jax.numpy as jnp
from jax import lax
from jax.experimental import pallas as pl
from jax.experimental.pallas import tpu as pltpu
from jax.experimental.pallas import tpu_sc as plsc

N = 10000
E = 160000
HID = 256
NPAD = 10016
NB = 32
BR = 313
ACC_R = 320
CAP = 161792
BLKB = 1024
GCH = 64
PBLK = 6400
STG = 1408
ROW_BLK = 2504

_mesh = plsc.VectorSubcoreMesh(core_axis_name="c", subcore_axis_name="s")
_params = pltpu.CompilerParams(needs_layout_passes=False)


def _worker_id():
    return lax.axis_index("s") * 2 + lax.axis_index("c")


def _lane_bcast(v, j):
    return lax.gather(
        v, jnp.full((16, 1), j, jnp.int32),
        lax.GatherDimensionNumbers(offset_dims=(), collapsed_slice_dims=(0,),
                                   start_index_map=(0,)),
        (1,), mode=lax.GatherScatterMode.PROMISE_IN_BOUNDS)


def _prep_body(src_h, dst_h, el_src, el_dl, cnt, sbuf0, dbuf0, sbuf1, dbuf1,
               st_src, st_dl, cbuf, sem0, sem1):
    w = _worker_id()
    lo = w * BR
    nblk = E // PBLK
    bufs = ((sbuf0, dbuf0, sem0), (sbuf1, dbuf1, sem1))

    def fire(g, sb, db, sem):
        off = pl.multiple_of(g * PBLK, 8)
        pltpu.async_copy(src_h.at[pl.ds(off, PBLK)], sb, sem)
        pltpu.async_copy(dst_h.at[pl.ds(off, PBLK)], db, sem)

    def make_inner(sb, db):
        def half(off, ptr_v):
            d16 = db[pl.ds(off, 16)]
            s16 = sb[pl.ds(off, 16)]
            msk = (d16 >= lo) & (d16 < lo + BR)
            csum = plsc.cumsum(msk.astype(jnp.int32))
            pos = ptr_v + csum - 1
            plsc.store_scatter(st_src, [pos], s16, mask=msk)
            plsc.store_scatter(st_dl, [pos], d16 - lo, mask=msk)
            return ptr_v + _lane_bcast(csum, 15)

        def inner(c, carry):
            ptr_v, hptr = carry
            base = c * 320
            for i in range(20):
                ptr_v = half(pl.multiple_of(base + i * 16, 8), ptr_v)

            def do_flush(args):
                ptr_v, hptr = args
                hoff = pl.multiple_of(w * CAP + hptr, 8)
                pltpu.sync_copy(st_src.at[pl.ds(0, 1024)],
                                el_src.at[pl.ds(hoff, 1024)])
                pltpu.sync_copy(st_dl.at[pl.ds(0, 1024)],
                                el_dl.at[pl.ds(hoff, 1024)])
                for tmv in range(21):
                    st_src[pl.ds(tmv * 16, 16)] = st_src[pl.ds(1024 + tmv * 16, 16)]
                    st_dl[pl.ds(tmv * 16, 16)] = st_dl[pl.ds(1024 + tmv * 16, 16)]
                return ptr_v - 1024, hptr + 1024

            return lax.cond(jnp.any(ptr_v >= 1024), do_flush, lambda a: a,
                            (ptr_v, hptr))

        return inner

    def block(carry, bufpair, fire_next):
        sb, db, sem = bufpair
        pltpu.make_async_copy(src_h.at[pl.ds(0, PBLK)], sb, sem).wait()
        pltpu.make_async_copy(dst_h.at[pl.ds(0, PBLK)], db, sem).wait()
        if fire_next is not None:
            fire_next()
        return pl.loop(0, PBLK // 320, init_carry=carry)(make_inner(sb, db))

    fire(0, *bufs[0])

    def pair(g, carry):
        carry = block(carry, bufs[0], lambda: fire(2 * g + 1, *bufs[1]))
        carry = block(carry, bufs[1],
                      lambda: pl.when(2 * g + 2 < nblk)(
                          lambda: fire(2 * g + 2, *bufs[0])))
        return carry

    carry = pl.loop(0, nblk // 2,
                    init_carry=(jnp.zeros((16,), jnp.int32),
                                jnp.int32(0)))(pair)
    if nblk % 2:
        carry = block(carry, bufs[0], None)
    ptr_v, hptr = carry
    ptr = ptr_v[0]

    for t in range(4):
        st_src[pl.ds(ptr + t * 16, 16)] = jnp.zeros((16,), jnp.int32)
        st_dl[pl.ds(ptr + t * 16, 16)] = jnp.full((16,), BR, jnp.int32)
    hoff = pl.multiple_of(w * CAP + hptr, 8)
    pltpu.sync_copy(st_src.at[pl.ds(0, STG)], el_src.at[pl.ds(hoff, STG)])
    pltpu.sync_copy(st_dl.at[pl.ds(0, STG)], el_dl.at[pl.ds(hoff, STG)])
    cbuf[...] = jnp.broadcast_to(hptr + ptr, (16,))
    pltpu.sync_copy(cbuf, cnt.at[pl.ds(pl.multiple_of(w * 16, 8), 16)])


_prep = functools.partial(
    pl.kernel,
    compiler_params=_params,
    out_type=(
        jax.ShapeDtypeStruct((NB * CAP,), jnp.int32),
        jax.ShapeDtypeStruct((NB * CAP,), jnp.int32),
        jax.ShapeDtypeStruct((NB * 16,), jnp.int32),
    ),
    mesh=_mesh,
    scratch_types=[
        pltpu.VMEM((PBLK,), jnp.int32),
        pltpu.VMEM((PBLK,), jnp.int32),
        pltpu.VMEM((PBLK,), jnp.int32),
        pltpu.VMEM((PBLK,), jnp.int32),
        pltpu.VMEM((STG,), jnp.int32),
        pltpu.VMEM((STG,), jnp.int32),
        pltpu.VMEM((16,), jnp.int32),
        pltpu.SemaphoreType.DMA,
        pltpu.SemaphoreType.DMA,
    ],
)(_prep_body)


def _agg_body(d, h, el_src, el_dl, cnt, zeros, m, src_blk, dl_blk, rows0,
              rows1, cbuf, acc, sem0, sem1):
    w = _worker_id()
    iota = lax.iota(jnp.int32, 16)
    pltpu.sync_copy(zeros, acc)
    pltpu.sync_copy(cnt.at[pl.ds(pl.multiple_of(w * 16, 8), 16)], cbuf)
    n = jnp.max(cbuf[...])
    nch = (n + (GCH - 1)) // GCH
    cpb = BLKB // GCH

    def fire(c, buf, sem):
        idx = src_blk.at[pl.ds(pl.multiple_of(c * GCH, 8), GCH)]
        pltpu.async_copy(h.at[idx], buf, sem)

    def wait(buf, sem):
        pltpu.make_async_copy(h.at[src_blk.at[pl.ds(0, GCH)]], buf, sem).wait()

    def process(buf, c):
        def edge(e):
            lane = e % 16
            voff = pl.multiple_of(c * GCH + (e // 16) * 16, 8)
            vec = dl_blk[pl.ds(voff, 16)]
            bc = lax.gather(
                vec, jnp.broadcast_to(lane, (16, 1)),
                lax.GatherDimensionNumbers(offset_dims=(),
                                           collapsed_slice_dims=(0,),
                                           start_index_map=(0,)),
                (1,), mode=lax.GatherScatterMode.PROMISE_IN_BOUNDS)
            rowbase = bc * d + iota
            for k in range(d // 32):
                v16 = buf[e, pl.ds(k * 16, 16)]
                va = plsc.bitcast(v16 << 16, jnp.float32)
                vb = plsc.bitcast(v16 & jnp.int32(-65536), jnp.float32)
                plsc.addupdate_scatter(acc, [rowbase + (k * 16)], va)
                plsc.addupdate_scatter(acc, [rowbase + (128 + k * 16)], vb)

        plsc.parallel_loop(0, GCH, 1, unroll=4)(edge)

    def outer(g):
        boff = pl.multiple_of(w * CAP + g * BLKB, 8)
        pltpu.sync_copy(el_src.at[pl.ds(boff, BLKB)], src_blk)
        pltpu.sync_copy(el_dl.at[pl.ds(boff, BLKB)], dl_blk)
        ch_here = jnp.minimum(cpb, nch - g * cpb)
        fire(0, rows0, sem0)

        def pair(p):
            c0 = p * 2
            c1 = c0 + 1
            wait(rows0, sem0)
            pl.when(c1 < ch_here)(lambda: fire(c1, rows1, sem1))
            process(rows0, c0)

            @pl.when(c1 < ch_here)
            def _():
                wait(rows1, sem1)
                pl.when(c1 + 1 < ch_here)(lambda: fire(c1 + 1, rows0, sem0))
                process(rows1, c1)

        pl.loop(0, (ch_here + 1) // 2)(pair)

    pl.loop(0, (nch + cpb - 1) // cpb)(outer)
    doff = pl.multiple_of(w * BR * d, 8)
    pltpu.sync_copy(acc.at[pl.ds(0, BR * d)], m.at[pl.ds(doff, BR * d)])


def _make_agg(d):
    return functools.partial(
        pl.kernel,
        compiler_params=_params,
        out_type=jax.ShapeDtypeStruct((NPAD * d,), jnp.float32),
        mesh=_mesh,
        scratch_types=[
            pltpu.VMEM((BLKB,), jnp.int32),
            pltpu.VMEM((BLKB,), jnp.int32),
            pltpu.VMEM((GCH, d // 2), jnp.int32),
            pltpu.VMEM((GCH, d // 2), jnp.int32),
            pltpu.VMEM((16,), jnp.int32),
            pltpu.VMEM((ACC_R * d,), jnp.float32),
            pltpu.SemaphoreType.DMA,
            pltpu.SemaphoreType.DMA,
        ],
    )(functools.partial(_agg_body, d))


_agg256 = _make_agg(HID)


def _pack_words(acc):
    u = jax.lax.bitcast_convert_type(acc, jnp.int32)
    lsb = jax.lax.shift_right_logical(u, 16) & 1
    rb = jax.lax.shift_right_logical(u + 0x7FFF + lsb, 16)
    return rb[:, :128] | (rb[:, 128:] << 16)


def _dense_body(m_ref, h_ref, wr_ref, wt_ref, b_ref, o_ref, ob_ref, *, relu,
                rel_mm):
    if rel_mm:
        acc = jnp.dot(m_ref[...], wr_ref[...], preferred_element_type=jnp.float32)
    else:
        acc = m_ref[...]
    acc = acc + jnp.dot(h_ref[...], wt_ref[...], preferred_element_type=jnp.float32)
    acc = acc + b_ref[...]
    if relu:
        acc = jnp.maximum(acc, 0.0)
    o_ref[...] = acc
    if ob_ref is not None:
        ob_ref[...] = _pack_words(acc)


def _dense_layer(m, h, W_rel, W_root, b, relu, rel_mm=True, emit_bf16=True):
    n, k = h.shape
    mk = m.shape[1]
    out_shape = [jax.ShapeDtypeStruct((n, HID), jnp.float32)]
    out_specs = [pl.BlockSpec((ROW_BLK, HID), lambda i: (i, 0))]
    body = functools.partial(_dense_body, relu=relu, rel_mm=rel_mm)
    if emit_bf16:
        out_shape.append(jax.ShapeDtypeStruct((n, HID // 2), jnp.int32))
        out_specs.append(pl.BlockSpec((ROW_BLK, HID // 2), lambda i: (i, 0)))
    else:
        body = functools.partial(_dense_body, relu=relu, rel_mm=rel_mm,
                                 ob_ref=None)
    out = pl.pallas_call(
        body,
        grid=(n // ROW_BLK,),
        in_specs=[
            pl.BlockSpec((ROW_BLK, mk), lambda i: (i, 0)),
            pl.BlockSpec((ROW_BLK, k), lambda i: (i, 0)),
            pl.BlockSpec((mk, HID), lambda i: (0, 0)),
            pl.BlockSpec((k, HID), lambda i: (0, 0)),
            pl.BlockSpec((1, HID), lambda i: (0, 0)),
        ],
        out_specs=out_specs,
        out_shape=out_shape,
    )(m, h, W_rel, W_root, b.reshape(1, HID))
    return out if emit_bf16 else out[0]


def _mm_body(h_ref, w_ref, o_ref):
    acc = jnp.dot(h_ref[...], w_ref[...], preferred_element_type=jnp.float32)
    o_ref[...] = _pack_words(acc)


def _matmul_bf16(h, w):
    n, k = h.shape
    return pl.pallas_call(
        _mm_body,
        grid=(n // ROW_BLK,),
        in_specs=[
            pl.BlockSpec((ROW_BLK, k), lambda i: (i, 0)),
            pl.BlockSpec((k, HID), lambda i: (0, 0)),
        ],
        out_specs=pl.BlockSpec((ROW_BLK, HID // 2), lambda i: (i, 0)),
        out_shape=jax.ShapeDtypeStruct((n, HID // 2), jnp.int32),
    )(h, w)


def kernel(x, edge_index, W_rel0, b_rel0, W_root0, W_rel1, b_rel1, W_root1,
           W_rel2, b_rel2, W_root2, W_rel3, b_rel3, W_root3):
    h0 = jnp.zeros((NPAD, 8), jnp.float32).at[:N, :6].set(x[:, 4:10])
    wr0 = jnp.zeros((8, HID), jnp.float32).at[:6].set(W_rel0)
    wt0 = jnp.zeros((8, HID), jnp.float32).at[:6].set(W_root0)
    z256 = jnp.zeros((ACC_R * HID,), jnp.float32)

    el_src, el_dl, cnt = _prep(edge_index[0], edge_index[1])

    t0 = _matmul_bf16(h0, wr0)
    m = _agg256(t0, el_src, el_dl, cnt, z256).reshape(NPAD, HID)
    h, hb = _dense_layer(m, h0, wr0, wt0, b_rel0, relu=True, rel_mm=False)
    for W_rel, b_rel, W_root, relu, last in (
        (W_rel1, b_rel1, W_root1, True, False),
        (W_rel2, b_rel2, W_root2, True, False),
        (W_rel3, b_rel3, W_root3, False, True),
    ):
        m = _agg256(hb, el_src, el_dl, cnt, z256).reshape(NPAD, HID)
        if last:
            h = _dense_layer(m, h, W_rel, W_root, b_rel, relu=False,
                             emit_bf16=False)
        else:
            h, hb = _dense_layer(m, h, W_rel, W_root, b_rel, relu=True)
    return h[:N]
